# baseline (speedup 1.0000x reference)
import functools

import jax
import jax.numpy as jnp
import numpy as np
from jax import lax
from jax.experimental import pallas as pl
from jax.experimental.pallas import tpu as pltpu
from jax.experimental.pallas import tpu_sc as plsc

B = 16384
H, W = 12, 42
ROW = 2 * H * W
BOARD = H * W
BOARDP = 512
BDIM = 14
EDIM = 7
OL = 5 * 7 * 7
OLP = 248
OE = EDIM * 7
OEP = 56
NW = 32
NB = B // NW
CB = 4
NCHUNK = NB // CB


def _local_perm() -> np.ndarray:
    idx = np.full((256,), BOARD, np.int32)
    for k in range(OL):
        i, r, c = k // 49, (k % 49) // 7, k % 7
        idx[k] = BOARD + r * 42 + 7 * i + c
    return idx


def _extra_consts() -> tuple[np.ndarray, np.ndarray]:
    jv = np.full((64,), BOARD + 35, np.int32)
    dv = np.zeros((64,), np.int32)
    for k in range(OE):
        jv[k] = BOARD + 35 + (k % 7)
        dv[k] = k // 7
    return jv, dv


_LP = _local_perm()
_EJ, _ED = _extra_consts()


@functools.cache
def _build_sc_parse():
    mesh = plsc.VectorSubcoreMesh(core_axis_name="c", subcore_axis_name="s")

    @functools.partial(
        pl.kernel,
        out_type=[
            jax.ShapeDtypeStruct((B, BDIM, BOARD), jnp.float32),
            jax.ShapeDtypeStruct((B, OLP), jnp.float32),
            jax.ShapeDtypeStruct((B, OEP), jnp.float32),
        ],
        mesh=mesh,
        compiler_params=pltpu.CompilerParams(
            needs_layout_passes=False, use_tc_tiling_on_sc=False,
            disable_bounds_checks=True),
        scratch_types=[
            pltpu.VMEM((2, CB, ROW), jnp.float32),
            pltpu.VMEM((10, BDIM), jnp.float32),
            pltpu.VMEM((20, EDIM), jnp.float32),
            pltpu.VMEM((256,), jnp.int32),
            pltpu.VMEM((64,), jnp.int32),
            pltpu.VMEM((64,), jnp.int32),
            pltpu.VMEM((2, CB, BDIM, BOARDP), jnp.float32),
            pltpu.VMEM((2, CB, 256), jnp.float32),
            pltpu.VMEM((2, CB, 64), jnp.float32),
            pltpu.SemaphoreType.DMA,
            pltpu.SemaphoreType.DMA,
            pltpu.SemaphoreType.DMA,
            pltpu.SemaphoreType.DMA,
            pltpu.SemaphoreType.DMA,
            pltpu.SemaphoreType.DMA,
            pltpu.SemaphoreType.DMA,
            pltpu.SemaphoreType.DMA,
        ],
    )
    def _sc_parse(xf, bt, et, lp, ej, ed, og, ol, oe,
                  xin, btv, etv, lpv, ejv, edv, ogb, olb, oeb,
                  isem0, isem1, gsem0, gsem1, lsem0, lsem1, esem0, esem1):
        isems = (isem0, isem1)
        gsems = (gsem0, gsem1)
        lsems = (lsem0, lsem1)
        esems = (esem0, esem1)
        wid = lax.axis_index("s") * 2 + lax.axis_index("c")
        base0 = wid * NB
        pltpu.sync_copy(bt, btv)
        pltpu.sync_copy(et, etv)
        pltpu.sync_copy(lp, lpv)
        pltpu.sync_copy(ej, ejv)
        pltpu.sync_copy(ed, edv)

        def start_in(sl, base):
            pltpu.async_copy(xf.at[pl.ds(base, CB)], xin.at[sl], isems[sl])

        def wait_in(sl, base):
            pltpu.make_async_copy(
                xf.at[pl.ds(base, CB)], xin.at[sl], isems[sl]).wait()

        def out_parts(sl, base):
            return (
                (ogb.at[sl, :, :, pl.ds(0, BOARD)],
                 og.at[pl.ds(base, CB)], gsems[sl]),
                (olb.at[sl, :, pl.ds(0, OLP)],
                 ol.at[pl.ds(base, CB)], lsems[sl]),
                (oeb.at[sl, :, pl.ds(0, OEP)],
                 oe.at[pl.ds(base, CB)], esems[sl]),
            )

        start_in(0, base0)
        start_in(1, base0 + CB)

        def chunk_pair(g, carry):
            for sl in (0, 1):
                chunk = g * 2 + sl
                base = base0 + chunk * CB
                wait_in(sl, base)

                @pl.when(chunk >= 2)
                def _drain():
                    for src, dst, sem in out_parts(sl, base):
                        pltpu.make_async_copy(src, dst, sem).wait()

                @plsc.parallel_loop(0, CB * 32, unroll=4)
                def _board(i):
                    bi = i >> 5
                    pv = i & 31
                    bidx = xin[sl, bi, pl.ds(pv * 16, 16)].astype(jnp.int32)
                    for d in range(BDIM):
                        vals = plsc.load_gather(
                            btv, [bidx, jnp.full((16,), d, jnp.int32)])
                        ogb[sl, bi, d, pl.ds(pv * 16, 16)] = vals

                @plsc.parallel_loop(0, CB, unroll=2)
                def _aux(bi):
                    bvec = jnp.full((16,), bi, jnp.int32)
                    for k in range(16):
                        src = plsc.load_gather(
                            xin.at[sl], [bvec, lpv[pl.ds(16 * k, 16)]])
                        olb[sl, bi, pl.ds(16 * k, 16)] = src
                    for k in range(4):
                        ix = plsc.load_gather(
                            xin.at[sl],
                            [bvec, ejv[pl.ds(16 * k, 16)]]).astype(jnp.int32)
                        vals = plsc.load_gather(
                            etv, [ix, edv[pl.ds(16 * k, 16)]])
                        oeb[sl, bi, pl.ds(16 * k, 16)] = vals

                for src, dst, sem in out_parts(sl, base):
                    pltpu.async_copy(src, dst, sem)

                @pl.when(chunk + 2 < NCHUNK)
                def _refill():
                    start_in(sl, base + 2 * CB)
            return carry

        lax.fori_loop(0, NCHUNK // 2, chunk_pair, 0)
        for sl in (0, 1):
            base = base0 + (NCHUNK - 2 + sl) * CB
            for src, dst, sem in out_parts(sl, base):
                pltpu.make_async_copy(src, dst, sem).wait()

    return _sc_parse


def kernel(x, board_table, extra_table):
    xf = x.reshape(B, ROW)
    og, ol, oe = _build_sc_parse()(
        xf, board_table, extra_table,
        jnp.asarray(_LP), jnp.asarray(_EJ), jnp.asarray(_ED))
    return (og.reshape(B, BDIM, H, W),
            ol[:, :OL].reshape(B, 5, 7, 7),
            oe[:, :OE].reshape(B, EDIM, 7))


# speedup vs baseline: 1.0643x; 1.0643x over previous
import functools

import jax
import jax.numpy as jnp
import numpy as np
from jax import lax
from jax.experimental import pallas as pl
from jax.experimental.pallas import tpu as pltpu
from jax.experimental.pallas import tpu_sc as plsc

B = 16384
H, W = 12, 42
ROW = 2 * H * W
BOARD = H * W
BOARDP = 512
BDIM = 14
EDIM = 7
OL = 5 * 7 * 7
OLP = 248
OE = EDIM * 7
OEP = 56
NW = 32
NB = B // NW
CB = 4
NCHUNK = NB // CB


def _local_perm() -> np.ndarray:
    idx = np.full((256,), BOARD, np.int32)
    for k in range(OL):
        i, r, c = k // 49, (k % 49) // 7, k % 7
        idx[k] = BOARD + r * 42 + 7 * i + c
    return idx


def _extra_consts() -> tuple[np.ndarray, np.ndarray]:
    jv = np.full((64,), BOARD + 35, np.int32)
    dv = np.zeros((64,), np.int32)
    for k in range(OE):
        jv[k] = BOARD + 35 + (k % 7)
        dv[k] = k // 7
    return jv, dv


_LP = _local_perm()
_EJ, _ED = _extra_consts()


@functools.cache
def _build_sc_parse():
    mesh = plsc.VectorSubcoreMesh(core_axis_name="c", subcore_axis_name="s")

    @functools.partial(
        pl.kernel,
        out_type=[
            jax.ShapeDtypeStruct((B, BDIM, BOARD), jnp.float32),
            jax.ShapeDtypeStruct((B, OLP), jnp.float32),
            jax.ShapeDtypeStruct((B, OEP), jnp.float32),
        ],
        mesh=mesh,
        compiler_params=pltpu.CompilerParams(
            needs_layout_passes=False, use_tc_tiling_on_sc=False,
            disable_bounds_checks=True),
        scratch_types=[
            pltpu.VMEM((2, CB, ROW), jnp.float32),
            pltpu.VMEM((10, BDIM), jnp.float32),
            pltpu.VMEM((20, EDIM), jnp.float32),
            pltpu.VMEM((256,), jnp.int32),
            pltpu.VMEM((64,), jnp.int32),
            pltpu.VMEM((64,), jnp.int32),
            pltpu.VMEM((2, CB, BDIM, BOARDP), jnp.float32),
            pltpu.VMEM((2, CB, 256), jnp.float32),
            pltpu.VMEM((2, CB, 64), jnp.float32),
            pltpu.SemaphoreType.DMA,
            pltpu.SemaphoreType.DMA,
            pltpu.SemaphoreType.DMA,
            pltpu.SemaphoreType.DMA,
            pltpu.SemaphoreType.DMA,
            pltpu.SemaphoreType.DMA,
            pltpu.SemaphoreType.DMA,
            pltpu.SemaphoreType.DMA,
        ],
    )
    def _sc_parse(xf, bt, et, lp, ej, ed, og, ol, oe,
                  xin, btv, etv, lpv, ejv, edv, ogb, olb, oeb,
                  isem0, isem1, gsem0, gsem1, lsem0, lsem1, esem0, esem1):
        isems = (isem0, isem1)
        gsems = (gsem0, gsem1)
        lsems = (lsem0, lsem1)
        esems = (esem0, esem1)
        wid = lax.axis_index("s") * 2 + lax.axis_index("c")
        base0 = wid * NB
        pltpu.sync_copy(bt, btv)
        pltpu.sync_copy(et, etv)
        pltpu.sync_copy(lp, lpv)
        pltpu.sync_copy(ej, ejv)
        pltpu.sync_copy(ed, edv)

        def start_in(sl, base):
            pltpu.async_copy(xf.at[pl.ds(base, CB)], xin.at[sl], isems[sl])

        def wait_in(sl, base):
            pltpu.make_async_copy(
                xf.at[pl.ds(base, CB)], xin.at[sl], isems[sl]).wait()

        def out_parts(sl, base):
            return (
                (ogb.at[sl, :, :, pl.ds(0, BOARD)],
                 og.at[pl.ds(base, CB)], gsems[sl]),
                (olb.at[sl, :, pl.ds(0, OLP)],
                 ol.at[pl.ds(base, CB)], lsems[sl]),
                (oeb.at[sl, :, pl.ds(0, OEP)],
                 oe.at[pl.ds(base, CB)], esems[sl]),
            )

        start_in(0, base0)
        start_in(1, base0 + CB)

        def chunk_pair(g, carry):
            for sl in (0, 1):
                chunk = g * 2 + sl
                base = base0 + chunk * CB
                wait_in(sl, base)

                @pl.when(chunk >= 2)
                def _drain():
                    for src, dst, sem in out_parts(sl, base):
                        pltpu.make_async_copy(src, dst, sem).wait()

                @plsc.parallel_loop(0, CB * 32, unroll=2)
                def _board(i):
                    bi = i >> 5
                    pv = i & 31
                    bidx = xin[sl, bi, pl.ds(pv * 16, 16)].astype(jnp.int32)
                    for d in range(BDIM):
                        vals = plsc.load_gather(
                            btv, [bidx, jnp.full((16,), d, jnp.int32)])
                        ogb[sl, bi, d, pl.ds(pv * 16, 16)] = vals

                @plsc.parallel_loop(0, CB, unroll=2)
                def _aux(bi):
                    bvec = jnp.full((16,), bi, jnp.int32)
                    for k in range(16):
                        src = plsc.load_gather(
                            xin.at[sl], [bvec, lpv[pl.ds(16 * k, 16)]])
                        olb[sl, bi, pl.ds(16 * k, 16)] = src
                    for k in range(4):
                        ix = plsc.load_gather(
                            xin.at[sl],
                            [bvec, ejv[pl.ds(16 * k, 16)]]).astype(jnp.int32)
                        vals = plsc.load_gather(
                            etv, [ix, edv[pl.ds(16 * k, 16)]])
                        oeb[sl, bi, pl.ds(16 * k, 16)] = vals

                for src, dst, sem in out_parts(sl, base):
                    pltpu.async_copy(src, dst, sem)

                @pl.when(chunk + 2 < NCHUNK)
                def _refill():
                    start_in(sl, base + 2 * CB)
            return carry

        lax.fori_loop(0, NCHUNK // 2, chunk_pair, 0)
        for sl in (0, 1):
            base = base0 + (NCHUNK - 2 + sl) * CB
            for src, dst, sem in out_parts(sl, base):
                pltpu.make_async_copy(src, dst, sem).wait()

    return _sc_parse


def kernel(x, board_table, extra_table):
    xf = x.reshape(B, ROW)
    og, ol, oe = _build_sc_parse()(
        xf, board_table, extra_table,
        jnp.asarray(_LP), jnp.asarray(_EJ), jnp.asarray(_ED))
    return (og.reshape(B, BDIM, H, W),
            ol[:, :OL].reshape(B, 5, 7, 7),
            oe[:, :OE].reshape(B, EDIM, 7))
